# baseline (speedup 1.0000x reference)
import jax
import jax.numpy as jnp
from jax.experimental import pallas as pl
from jax.experimental.pallas import tpu as pltpu

B = 16384
D = 128
R = 128


def _mlp_kernel(xvc_ref, w1_ref, b1_ref, w2_ref, b2_ref, oc_ref):
    x = xvc_ref[...]
    b2s = b2_ref[0, 0]
    acc = jnp.full((R, 128), b2s, dtype=jnp.float32)
    for d in range(D):
        t = jnp.maximum(x * w1_ref[0, d] + b1_ref[0, d], 0.0)
        tb = t.astype(jnp.bfloat16).astype(jnp.float32)
        acc = acc + tb * w2_ref[0, d]
    oc_ref[0] = jnp.full((R, 128), b2s, dtype=jnp.float32)
    oc_ref[1] = acc


def kernel(xv, xi, emb, W1, b1, W2, b2):
    xvc = xv.reshape(R, 128)
    w1 = W1.reshape(1, D)
    b1r = b1.reshape(1, D)
    w2 = W2.astype(jnp.bfloat16).astype(jnp.float32).reshape(1, D)
    b2r = b2.reshape(1, 1)

    oc = pl.pallas_call(
        _mlp_kernel,
        grid=(1,),
        in_specs=[
            pl.BlockSpec((R, 128), lambda i: (0, 0)),
            pl.BlockSpec(memory_space=pltpu.SMEM),
            pl.BlockSpec(memory_space=pltpu.SMEM),
            pl.BlockSpec(memory_space=pltpu.SMEM),
            pl.BlockSpec(memory_space=pltpu.SMEM),
        ],
        out_specs=pl.BlockSpec((2, R, 128), lambda i: (0, 0, 0)),
        out_shape=jax.ShapeDtypeStruct((2, R, 128), jnp.float32),
    )(xvc, w1, b1r, w2, b2r)
    return oc.reshape(2 * B, 1)


# speedup vs baseline: 1.0022x; 1.0022x over previous
import jax
import jax.numpy as jnp
from jax.experimental import pallas as pl
from jax.experimental.pallas import tpu as pltpu

B = 16384
D = 128
R = 128


def _mlp_kernel(xvc_ref, w1_ref, b1_ref, w2_ref, b2_ref, oc_ref):
    x = xvc_ref[...]
    b2s = b2_ref[0, 0]
    acc = jnp.full((R, 128), b2s, dtype=jnp.float32)
    for d in range(D):
        t = jnp.maximum(x * w1_ref[0, d] + b1_ref[0, d], 0.0)
        tb = t.astype(jnp.bfloat16).astype(jnp.float32)
        acc = acc + tb * w2_ref[0, d]
    oc_ref[0] = jnp.full((R, 128), b2s, dtype=jnp.float32)
    oc_ref[1] = acc


def kernel(xv, xi, emb, W1, b1, W2, b2):
    xvc = xv.reshape(R, 128)
    w1 = W1.reshape(1, D)
    b1r = b1.reshape(1, D)
    w2bits = jax.lax.bitcast_convert_type(W2.reshape(1, D), jnp.uint32)
    lsb = jax.lax.shift_right_logical(w2bits, jnp.uint32(16)) & jnp.uint32(1)
    w2bits = (w2bits + jnp.uint32(0x7FFF) + lsb) & jnp.uint32(0xFFFF0000)
    w2 = jax.lax.bitcast_convert_type(w2bits, jnp.float32)
    b2r = b2.reshape(1, 1)

    oc = pl.pallas_call(
        _mlp_kernel,
        grid=(1,),
        in_specs=[
            pl.BlockSpec((R, 128), lambda i: (0, 0)),
            pl.BlockSpec(memory_space=pltpu.SMEM),
            pl.BlockSpec(memory_space=pltpu.SMEM),
            pl.BlockSpec(memory_space=pltpu.SMEM),
            pl.BlockSpec(memory_space=pltpu.SMEM),
        ],
        out_specs=pl.BlockSpec((2, R, 128), lambda i: (0, 0, 0)),
        out_shape=jax.ShapeDtypeStruct((2, R, 128), jnp.float32),
    )(xvc, w1, b1r, w2, b2r)
    return oc.reshape(2 * B, 1)
